# baseline (speedup 1.0000x reference)
import jax
import jax.numpy as jnp
from jax import lax
from jax.experimental import pallas as pl
from jax.experimental.pallas import tpu as pltpu
from jax.experimental.pallas import tpu_sc as plsc

NUM_CORES = 2
NUM_SUBCORES = 16
LANES = 16
NW = NUM_CORES * NUM_SUBCORES

BATCH = 16384
EMB_DIM = 32
B_PER_W = BATCH // NW
IDX_CHUNK = 128
N_CHUNKS = B_PER_W // IDX_CHUNK
N_GROUPS = B_PER_W // LANES
BURST = 8


def _mf_kernel(user_hbm, item_hbm, userf_hbm, itemf_hbm, utab_hbm, itab_hbm,
               ubias_hbm, ibias_hbm, out_hbm, uidx_v, iidx_v, uflat_v, iflat_v,
               utile_v, itile_v, ucols_v, icols_v, ub_v, ib_v, out_v, sem,
               bsem):
  wid = lax.axis_index("s") * NUM_CORES + lax.axis_index("c")
  base = wid * B_PER_W

  pltpu.sync_copy(user_hbm.at[wid], uidx_v)
  pltpu.sync_copy(item_hbm.at[wid], iidx_v)
  pltpu.sync_copy(userf_hbm.at[wid], uflat_v)
  pltpu.sync_copy(itemf_hbm.at[wid], iflat_v)

  bias_copies = []
  for j in range(N_CHUNKS):
    dst = pl.ds(j * IDX_CHUNK, IDX_CHUNK)
    bias_copies.append(pltpu.async_copy(ubias_hbm.at[uidx_v.at[j]],
                                        ub_v.at[dst], bsem))
    bias_copies.append(pltpu.async_copy(ibias_hbm.at[iidx_v.at[j]],
                                        ib_v.at[dst], bsem))

  rows_lo = lax.iota(jnp.int32, LANES)
  rows_hi = rows_lo + LANES

  def group_body(g, carry):
    s = pl.ds(g * LANES, LANES)
    uvec = uflat_v[s]
    ivec = iflat_v[s]
    ucs = (uvec >> 7) << 7
    ics = (ivec >> 7) << 7
    umod = uvec & 127
    imod = ivec & 127

    for p in range(LANES // BURST):
      copies = []
      for t in range(BURST):
        lk = p * BURST + t
        cu = pl.multiple_of(ucs[lk], 128)
        copies.append(pltpu.async_copy(utab_hbm.at[:, pl.ds(cu, 128)],
                                       utile_v.at[t], sem))
        ci = pl.multiple_of(ics[lk], 128)
        copies.append(pltpu.async_copy(itab_hbm.at[:, pl.ds(ci, 128)],
                                       itile_v.at[t], sem))
      for c in copies:
        c.wait()
      for t in range(BURST):
        lk = p * BURST + t
        bsel = jnp.full((LANES,), t, jnp.int32)
        mu = jnp.full((LANES,), 0, jnp.int32) + umod[lk]
        mi = jnp.full((LANES,), 0, jnp.int32) + imod[lk]
        col = jnp.full((LANES,), lk, jnp.int32)
        for rows in (rows_lo, rows_hi):
          plsc.store_scatter(ucols_v, [rows, col],
                             plsc.load_gather(utile_v, [bsel, rows, mu]))
          plsc.store_scatter(icols_v, [rows, col],
                             plsc.load_gather(itile_v, [bsel, rows, mi]))

    acc = ucols_v[0, pl.ds(0, LANES)] * icols_v[0, pl.ds(0, LANES)]
    for d in range(1, EMB_DIM):
      acc = acc + ucols_v[d, pl.ds(0, LANES)] * icols_v[d, pl.ds(0, LANES)]
    out_v[s] = acc
    return carry

  lax.fori_loop(0, N_GROUPS, group_body, 0)

  for c in bias_copies:
    c.wait()

  def bias_body(g, carry):
    s = pl.ds(g * LANES, LANES)
    out_v[s] = out_v[s] + ub_v[s] + ib_v[s]
    return carry

  lax.fori_loop(0, N_GROUPS, bias_body, 0)

  pltpu.sync_copy(out_v, out_hbm.at[pl.ds(base, B_PER_W)])


@jax.jit
def kernel(user, item, user_table, item_table, user_bias_table,
           item_bias_table):
  user3 = user.astype(jnp.int32).reshape(NW, N_CHUNKS, IDX_CHUNK)
  item3 = item.astype(jnp.int32).reshape(NW, N_CHUNKS, IDX_CHUNK)
  user2 = user.astype(jnp.int32).reshape(NW, B_PER_W)
  item2 = item.astype(jnp.int32).reshape(NW, B_PER_W)

  mesh = plsc.VectorSubcoreMesh(core_axis_name="c", subcore_axis_name="s",
                                num_cores=NUM_CORES,
                                num_subcores=NUM_SUBCORES)
  run = pl.kernel(
      _mf_kernel,
      out_type=jax.ShapeDtypeStruct((BATCH,), jnp.float32),
      mesh=mesh,
      compiler_params=pltpu.CompilerParams(needs_layout_passes=False),
      scratch_types=[
          pltpu.VMEM((N_CHUNKS, IDX_CHUNK), jnp.int32),
          pltpu.VMEM((N_CHUNKS, IDX_CHUNK), jnp.int32),
          pltpu.VMEM((B_PER_W,), jnp.int32),
          pltpu.VMEM((B_PER_W,), jnp.int32),
          pltpu.VMEM((BURST, EMB_DIM, 128), jnp.float32),
          pltpu.VMEM((BURST, EMB_DIM, 128), jnp.float32),
          pltpu.VMEM((EMB_DIM, LANES), jnp.float32),
          pltpu.VMEM((EMB_DIM, LANES), jnp.float32),
          pltpu.VMEM((B_PER_W,), jnp.float32),
          pltpu.VMEM((B_PER_W,), jnp.float32),
          pltpu.VMEM((B_PER_W,), jnp.float32),
          pltpu.SemaphoreType.DMA,
          pltpu.SemaphoreType.DMA,
      ],
  )
  return run(user3, item3, user2, item2, user_table.T, item_table.T,
             user_bias_table.reshape(-1), item_bias_table.reshape(-1))


# speedup vs baseline: 1.0110x; 1.0110x over previous
import jax
import jax.numpy as jnp
from jax import lax
from jax.experimental import pallas as pl
from jax.experimental.pallas import tpu as pltpu
from jax.experimental.pallas import tpu_sc as plsc

NUM_CORES = 2
NUM_SUBCORES = 16
LANES = 16
NW = NUM_CORES * NUM_SUBCORES

BATCH = 16384
EMB_DIM = 32
B_PER_W = BATCH // NW
IDX_CHUNK = 128
N_CHUNKS = B_PER_W // IDX_CHUNK
N_GROUPS = B_PER_W // LANES
BURST = 4


def _mf_kernel(user_hbm, item_hbm, userf_hbm, itemf_hbm, utab_hbm, itab_hbm,
               ubias_hbm, ibias_hbm, out_hbm, uidx_v, iidx_v, uflat_v, iflat_v,
               utile_v, itile_v, ucols_v, icols_v, ub_v, ib_v, out_v, sem_a,
               sem_b, bsem):
  sems = (sem_a, sem_b)
  wid = lax.axis_index("s") * NUM_CORES + lax.axis_index("c")
  base = wid * B_PER_W

  pltpu.sync_copy(user_hbm.at[wid], uidx_v)
  pltpu.sync_copy(item_hbm.at[wid], iidx_v)
  pltpu.sync_copy(userf_hbm.at[wid], uflat_v)
  pltpu.sync_copy(itemf_hbm.at[wid], iflat_v)

  bias_copies = []
  for j in range(N_CHUNKS):
    dst = pl.ds(j * IDX_CHUNK, IDX_CHUNK)
    bias_copies.append(pltpu.async_copy(ubias_hbm.at[uidx_v.at[j]],
                                        ub_v.at[dst], bsem))
    bias_copies.append(pltpu.async_copy(ibias_hbm.at[iidx_v.at[j]],
                                        ib_v.at[dst], bsem))

  rows_lo = lax.iota(jnp.int32, LANES)
  rows_hi = rows_lo + LANES
  N_BURSTS = LANES // BURST

  def group_body(g, carry):
    s = pl.ds(g * LANES, LANES)
    uvec = uflat_v[s]
    ivec = iflat_v[s]
    ucs = (uvec >> 7) << 7
    ics = (ivec >> 7) << 7
    umod = uvec & 127
    imod = ivec & 127

    def fire(b):
      sl = b & 1
      cs = []
      for t in range(BURST):
        lk = b * BURST + t
        cu = pl.multiple_of(ucs[lk], 128)
        cs.append(pltpu.async_copy(utab_hbm.at[:, pl.ds(cu, 128)],
                                   utile_v.at[sl, t], sems[sl]))
        ci = pl.multiple_of(ics[lk], 128)
        cs.append(pltpu.async_copy(itab_hbm.at[:, pl.ds(ci, 128)],
                                   itile_v.at[sl, t], sems[sl]))
      return cs

    def extract(b, cs):
      sl = b & 1
      for c in cs:
        c.wait()
      bv = jnp.full((LANES,), sl, jnp.int32)
      for t in range(BURST):
        lk = b * BURST + t
        tsel = jnp.full((LANES,), t, jnp.int32)
        mu = jnp.full((LANES,), 0, jnp.int32) + umod[lk]
        mi = jnp.full((LANES,), 0, jnp.int32) + imod[lk]
        col = jnp.full((LANES,), lk, jnp.int32)
        for rows in (rows_lo, rows_hi):
          plsc.store_scatter(ucols_v, [rows, col],
                             plsc.load_gather(utile_v, [bv, tsel, rows, mu]))
          plsc.store_scatter(icols_v, [rows, col],
                             plsc.load_gather(itile_v, [bv, tsel, rows, mi]))

    pending = fire(0)
    for b in range(N_BURSTS):
      nxt = fire(b + 1) if b + 1 < N_BURSTS else None
      extract(b, pending)
      pending = nxt

    acc = ucols_v[0, pl.ds(0, LANES)] * icols_v[0, pl.ds(0, LANES)]
    for d in range(1, EMB_DIM):
      acc = acc + ucols_v[d, pl.ds(0, LANES)] * icols_v[d, pl.ds(0, LANES)]
    out_v[s] = acc
    return carry

  lax.fori_loop(0, N_GROUPS, group_body, 0)

  for c in bias_copies:
    c.wait()

  def bias_body(g, carry):
    s = pl.ds(g * LANES, LANES)
    out_v[s] = out_v[s] + ub_v[s] + ib_v[s]
    return carry

  lax.fori_loop(0, N_GROUPS, bias_body, 0)

  pltpu.sync_copy(out_v, out_hbm.at[pl.ds(base, B_PER_W)])


@jax.jit
def kernel(user, item, user_table, item_table, user_bias_table,
           item_bias_table):
  user3 = user.astype(jnp.int32).reshape(NW, N_CHUNKS, IDX_CHUNK)
  item3 = item.astype(jnp.int32).reshape(NW, N_CHUNKS, IDX_CHUNK)
  user2 = user.astype(jnp.int32).reshape(NW, B_PER_W)
  item2 = item.astype(jnp.int32).reshape(NW, B_PER_W)

  mesh = plsc.VectorSubcoreMesh(core_axis_name="c", subcore_axis_name="s",
                                num_cores=NUM_CORES,
                                num_subcores=NUM_SUBCORES)
  run = pl.kernel(
      _mf_kernel,
      out_type=jax.ShapeDtypeStruct((BATCH,), jnp.float32),
      mesh=mesh,
      compiler_params=pltpu.CompilerParams(needs_layout_passes=False),
      scratch_types=[
          pltpu.VMEM((N_CHUNKS, IDX_CHUNK), jnp.int32),
          pltpu.VMEM((N_CHUNKS, IDX_CHUNK), jnp.int32),
          pltpu.VMEM((B_PER_W,), jnp.int32),
          pltpu.VMEM((B_PER_W,), jnp.int32),
          pltpu.VMEM((2, BURST, EMB_DIM, 128), jnp.float32),
          pltpu.VMEM((2, BURST, EMB_DIM, 128), jnp.float32),
          pltpu.VMEM((EMB_DIM, LANES), jnp.float32),
          pltpu.VMEM((EMB_DIM, LANES), jnp.float32),
          pltpu.VMEM((B_PER_W,), jnp.float32),
          pltpu.VMEM((B_PER_W,), jnp.float32),
          pltpu.VMEM((B_PER_W,), jnp.float32),
          pltpu.SemaphoreType.DMA,
          pltpu.SemaphoreType.DMA,
          pltpu.SemaphoreType.DMA,
      ],
  )
  return run(user3, item3, user2, item2, user_table.T, item_table.T,
             user_bias_table.reshape(-1), item_bias_table.reshape(-1))


# speedup vs baseline: 1.3087x; 1.2944x over previous
import jax
import jax.numpy as jnp
from jax import lax
from jax.experimental import pallas as pl
from jax.experimental.pallas import tpu as pltpu
from jax.experimental.pallas import tpu_sc as plsc

NUM_CORES = 2
NUM_SUBCORES = 16
LANES = 16
NW = NUM_CORES * NUM_SUBCORES

BATCH = 16384
EMB_DIM = 32
B_PER_W = BATCH // NW
IDX_CHUNK = 128
N_CHUNKS = B_PER_W // IDX_CHUNK
N_GROUPS = B_PER_W // LANES
BURST = 4


def _mf_kernel(userf_hbm, itemf_hbm, utab_hbm, itab_hbm, out_hbm, uflat_v,
               iflat_v, utile_v, itile_v, ucols_v, icols_v, out_v, sem_a,
               sem_b):
  sems = (sem_a, sem_b)
  wid = lax.axis_index("s") * NUM_CORES + lax.axis_index("c")
  base = wid * B_PER_W

  pltpu.sync_copy(userf_hbm.at[wid], uflat_v)
  pltpu.sync_copy(itemf_hbm.at[wid], iflat_v)

  rows_lo = lax.iota(jnp.int32, LANES)
  rows_hi = rows_lo + LANES
  N_BURSTS = LANES // BURST

  def group_body(g, carry):
    s = pl.ds(g * LANES, LANES)
    uvec = uflat_v[s]
    ivec = iflat_v[s]
    ucs = (uvec >> 7) << 7
    ics = (ivec >> 7) << 7
    umod = uvec & 127
    imod = ivec & 127

    def fire(b):
      sl = b & 1
      cs = []
      for t in range(BURST):
        lk = b * BURST + t
        cu = pl.multiple_of(ucs[lk], 128)
        cs.append(pltpu.async_copy(utab_hbm.at[:, pl.ds(cu, 128)],
                                   utile_v.at[sl, t], sems[sl]))
        ci = pl.multiple_of(ics[lk], 128)
        cs.append(pltpu.async_copy(itab_hbm.at[:, pl.ds(ci, 128)],
                                   itile_v.at[sl, t], sems[sl]))
      return cs

    def extract(b, cs):
      sl = b & 1
      for c in cs:
        c.wait()
      bv = jnp.full((LANES,), sl, jnp.int32)
      for t in range(BURST):
        lk = b * BURST + t
        tsel = jnp.full((LANES,), t, jnp.int32)
        mu = jnp.full((LANES,), 0, jnp.int32) + umod[lk]
        mi = jnp.full((LANES,), 0, jnp.int32) + imod[lk]
        col = jnp.full((LANES,), lk, jnp.int32)
        for rows in (rows_lo, rows_hi):
          plsc.store_scatter(ucols_v, [rows, col],
                             plsc.load_gather(utile_v, [bv, tsel, rows, mu]))
          plsc.store_scatter(icols_v, [rows, col],
                             plsc.load_gather(itile_v, [bv, tsel, rows, mi]))

    pending = fire(0)
    for b in range(N_BURSTS):
      nxt = fire(b + 1) if b + 1 < N_BURSTS else None
      extract(b, pending)
      pending = nxt

    acc = ucols_v[0, pl.ds(0, LANES)] * icols_v[0, pl.ds(0, LANES)]
    for d in range(1, EMB_DIM):
      acc = acc + ucols_v[d, pl.ds(0, LANES)] * icols_v[d, pl.ds(0, LANES)]
    out_v[s] = acc
    return carry

  lax.fori_loop(0, N_GROUPS, group_body, 0)

  pltpu.sync_copy(out_v, out_hbm.at[pl.ds(base, B_PER_W)])


def _bias_kernel(user_hbm, item_hbm, dots_hbm, ubias_hbm, ibias_hbm, out_hbm,
                 uidx_v, iidx_v, ub_v, ib_v, out_v, bsem):
  wid = lax.axis_index("s") * NUM_CORES + lax.axis_index("c")
  base = wid * B_PER_W

  pltpu.sync_copy(user_hbm.at[wid], uidx_v)
  pltpu.sync_copy(item_hbm.at[wid], iidx_v)
  pltpu.sync_copy(dots_hbm.at[pl.ds(base, B_PER_W)], out_v)

  bias_copies = []
  for j in range(N_CHUNKS):
    dst = pl.ds(j * IDX_CHUNK, IDX_CHUNK)
    bias_copies.append(pltpu.async_copy(ubias_hbm.at[uidx_v.at[j]],
                                        ub_v.at[dst], bsem))
    bias_copies.append(pltpu.async_copy(ibias_hbm.at[iidx_v.at[j]],
                                        ib_v.at[dst], bsem))
  for c in bias_copies:
    c.wait()

  def bias_body(g, carry):
    s = pl.ds(g * LANES, LANES)
    out_v[s] = out_v[s] + ub_v[s] + ib_v[s]
    return carry

  lax.fori_loop(0, N_GROUPS, bias_body, 0)

  pltpu.sync_copy(out_v, out_hbm.at[pl.ds(base, B_PER_W)])


@jax.jit
def kernel(user, item, user_table, item_table, user_bias_table,
           item_bias_table):
  user3 = user.astype(jnp.int32).reshape(NW, N_CHUNKS, IDX_CHUNK)
  item3 = item.astype(jnp.int32).reshape(NW, N_CHUNKS, IDX_CHUNK)
  user2 = user.astype(jnp.int32).reshape(NW, B_PER_W)
  item2 = item.astype(jnp.int32).reshape(NW, B_PER_W)

  mesh = plsc.VectorSubcoreMesh(core_axis_name="c", subcore_axis_name="s",
                                num_cores=NUM_CORES,
                                num_subcores=NUM_SUBCORES)
  run = pl.kernel(
      _mf_kernel,
      out_type=jax.ShapeDtypeStruct((BATCH,), jnp.float32),
      mesh=mesh,
      compiler_params=pltpu.CompilerParams(needs_layout_passes=False),
      scratch_types=[
          pltpu.VMEM((B_PER_W,), jnp.int32),
          pltpu.VMEM((B_PER_W,), jnp.int32),
          pltpu.VMEM((2, BURST, EMB_DIM, 128), jnp.float32),
          pltpu.VMEM((2, BURST, EMB_DIM, 128), jnp.float32),
          pltpu.VMEM((EMB_DIM, LANES), jnp.float32),
          pltpu.VMEM((EMB_DIM, LANES), jnp.float32),
          pltpu.VMEM((B_PER_W,), jnp.float32),
          pltpu.SemaphoreType.DMA,
          pltpu.SemaphoreType.DMA,
      ],
  )
  dots = run(user2, item2, user_table.T, item_table.T)

  run_bias = pl.kernel(
      _bias_kernel,
      out_type=jax.ShapeDtypeStruct((BATCH,), jnp.float32),
      mesh=mesh,
      compiler_params=pltpu.CompilerParams(needs_layout_passes=False),
      scratch_types=[
          pltpu.VMEM((N_CHUNKS, IDX_CHUNK), jnp.int32),
          pltpu.VMEM((N_CHUNKS, IDX_CHUNK), jnp.int32),
          pltpu.VMEM((B_PER_W,), jnp.float32),
          pltpu.VMEM((B_PER_W,), jnp.float32),
          pltpu.VMEM((B_PER_W,), jnp.float32),
          pltpu.SemaphoreType.DMA,
      ],
  )
  return run_bias(user3, item3, dots, user_bias_table.reshape(-1),
                  item_bias_table.reshape(-1))
